# Initial kernel scaffold; baseline (speedup 1.0000x reference)
import jax
import jax.numpy as jnp
from jax.experimental import pallas as pl


def kernel(seqs, emb_table, W):
    raise NotImplementedError("write your pallas kernel here")


# speedup vs baseline: 11.0074x; 11.0074x over previous
import functools

import jax
import jax.numpy as jnp
from jax import lax
from jax.experimental import pallas as pl
from jax.experimental.pallas import tpu as pltpu
from jax.experimental.pallas import tpu_sc as plsc

VOCAB = 1000000
EMB = 128
OUT = 64
B_TOTAL = 16384 * 50

NC, NS = 2, 16
NW = NC * NS
CH = 128
B_PER_W = B_TOTAL // NW
NCHUNK = B_PER_W // CH

PROJ_BLK = 8192


def _make_gather():
    mesh = plsc.VectorSubcoreMesh(core_axis_name="c", subcore_axis_name="s")

    @functools.partial(
        pl.kernel,
        mesh=mesh,
        out_type=jax.ShapeDtypeStruct((B_TOTAL, EMB), jnp.float32),
        scratch_types=[
            pltpu.VMEM((NCHUNK, CH), jnp.int32),
            pltpu.VMEM((CH, EMB), jnp.float32),
            pltpu.SemaphoreType.DMA,
        ],
    )
    def gather_k(tab_hbm, idx_hbm, out_hbm, idx_v, rows_v, gsem):
        wid = lax.axis_index("s") * NC + lax.axis_index("c")
        base = wid * B_PER_W
        pltpu.sync_copy(idx_hbm.at[wid], idx_v)

        def chunk(c, carry):
            pltpu.async_copy(tab_hbm.at[idx_v.at[c]], rows_v, gsem).wait()
            pltpu.sync_copy(rows_v, out_hbm.at[pl.ds(base + c * CH, CH)])
            return carry

        lax.fori_loop(0, NCHUNK, chunk, 0)

    return gather_k


_gather_rows = _make_gather()


def _proj_body(g_ref, w_ref, o_ref):
    o_ref[...] = lax.dot_general(
        g_ref[...], w_ref[...],
        dimension_numbers=(((1,), (1,)), ((), ())),
        preferred_element_type=jnp.float32,
    )


def _project(gathered, W):
    return pl.pallas_call(
        _proj_body,
        grid=(B_TOTAL // PROJ_BLK,),
        in_specs=[
            pl.BlockSpec((PROJ_BLK, EMB), lambda i: (i, 0)),
            pl.BlockSpec((OUT, EMB), lambda i: (0, 0)),
        ],
        out_specs=pl.BlockSpec((PROJ_BLK, OUT), lambda i: (i, 0)),
        out_shape=jax.ShapeDtypeStruct((B_TOTAL, OUT), jnp.float32),
    )(gathered, W)


def kernel(seqs, emb_table, W):
    idx = seqs.astype(jnp.int32).reshape(NW, NCHUNK, CH)
    gathered = _gather_rows(emb_table, idx)
    return _project(gathered, W).reshape(16384, 50, OUT)


# speedup vs baseline: 12.3461x; 1.1216x over previous
import functools

import jax
import jax.numpy as jnp
from jax import lax
from jax.experimental import pallas as pl
from jax.experimental.pallas import tpu as pltpu
from jax.experimental.pallas import tpu_sc as plsc

VOCAB = 1000000
EMB = 128
OUT = 64
B_TOTAL = 16384 * 50

NC, NS = 2, 16
NW = NC * NS
CH = 128
B_PER_W = B_TOTAL // NW
NCHUNK = B_PER_W // CH

PROJ_BLK = 8192


NBUF = 4
NGRP = NCHUNK // NBUF


def _make_gather():
    mesh = plsc.VectorSubcoreMesh(core_axis_name="c", subcore_axis_name="s")

    @functools.partial(
        pl.kernel,
        mesh=mesh,
        out_type=jax.ShapeDtypeStruct((B_TOTAL, EMB), jnp.float32),
        scratch_types=[
            pltpu.VMEM((NCHUNK, CH), jnp.int32),
            pltpu.VMEM((NBUF, CH, EMB), jnp.float32),
        ] + [pltpu.SemaphoreType.DMA] * (2 * NBUF),
    )
    def gather_k(tab_hbm, idx_hbm, out_hbm, idx_v, rows_v, *sems):
        gsems, osems = sems[:NBUF], sems[NBUF:]
        wid = lax.axis_index("s") * NC + lax.axis_index("c")
        base = wid * B_PER_W
        pltpu.sync_copy(idx_hbm.at[wid], idx_v)

        def gather_copy(c, b):
            return pltpu.make_async_copy(
                tab_hbm.at[idx_v.at[c]], rows_v.at[b], gsems[b])

        def out_copy(c, b):
            return pltpu.make_async_copy(
                rows_v.at[b], out_hbm.at[pl.ds(base + c * CH, CH)], osems[b])

        for b in range(NBUF):
            gather_copy(b, b).start()

        def group(g, carry):
            for b in range(NBUF):
                c = g * NBUF + b
                gather_copy(c, b).wait()
                out_copy(c, b).start()
            for b in range(NBUF):
                c = g * NBUF + b
                out_copy(c, b).wait()

                @pl.when(g < NGRP - 1)
                def _():
                    gather_copy(c + NBUF, b).start()

            return carry

        lax.fori_loop(0, NGRP, group, 0)

    return gather_k


_gather_rows = _make_gather()


def _proj_body(g_ref, w_ref, o_ref):
    o_ref[...] = lax.dot_general(
        g_ref[...], w_ref[...],
        dimension_numbers=(((1,), (1,)), ((), ())),
        preferred_element_type=jnp.float32,
    )


def _project(gathered, W):
    return pl.pallas_call(
        _proj_body,
        grid=(B_TOTAL // PROJ_BLK,),
        in_specs=[
            pl.BlockSpec((PROJ_BLK, EMB), lambda i: (i, 0)),
            pl.BlockSpec((OUT, EMB), lambda i: (0, 0)),
        ],
        out_specs=pl.BlockSpec((PROJ_BLK, OUT), lambda i: (i, 0)),
        out_shape=jax.ShapeDtypeStruct((B_TOTAL, OUT), jnp.float32),
    )(gathered, W)


def kernel(seqs, emb_table, W):
    idx = seqs.astype(jnp.int32).reshape(NW, NCHUNK, CH)
    gathered = _gather_rows(emb_table, idx)
    return _project(gathered, W).reshape(16384, 50, OUT)


# speedup vs baseline: 15.6516x; 1.2677x over previous
import functools

import jax
import jax.numpy as jnp
from jax import lax
from jax.experimental import pallas as pl
from jax.experimental.pallas import tpu as pltpu
from jax.experimental.pallas import tpu_sc as plsc

VOCAB = 1000000
EMB = 128
OUT = 64
B_TOTAL = 16384 * 50

NC, NS = 2, 16
NW = NC * NS
CH = 128
B_PER_W = B_TOTAL // NW
NCHUNK = B_PER_W // CH

BATCH = 16384
HIST = 50
PROJ_BB = 128


NBUF = 4
NGRP = NCHUNK // NBUF


def _make_gather():
    mesh = plsc.VectorSubcoreMesh(core_axis_name="c", subcore_axis_name="s")

    @functools.partial(
        pl.kernel,
        mesh=mesh,
        out_type=jax.ShapeDtypeStruct((B_TOTAL, EMB), jnp.float32),
        scratch_types=[
            pltpu.VMEM((NCHUNK, CH), jnp.int32),
            pltpu.VMEM((NBUF, CH, EMB), jnp.float32),
        ] + [pltpu.SemaphoreType.DMA] * (2 * NBUF),
    )
    def gather_k(tab_hbm, idx_hbm, out_hbm, idx_v, rows_v, *sems):
        gsems, osems = sems[:NBUF], sems[NBUF:]
        wid = lax.axis_index("s") * NC + lax.axis_index("c")
        base = wid * B_PER_W
        pltpu.sync_copy(idx_hbm.at[wid], idx_v)

        def gather_copy(c, b):
            return pltpu.make_async_copy(
                tab_hbm.at[idx_v.at[c]], rows_v.at[b], gsems[b])

        def out_copy(c, b):
            return pltpu.make_async_copy(
                rows_v.at[b], out_hbm.at[pl.ds(base + c * CH, CH)], osems[b])

        for b in range(NBUF):
            gather_copy(b, b).start()

        def group(g, carry):
            for b in range(NBUF):
                c = g * NBUF + b
                gather_copy(c, b).wait()
                out_copy(c, b).start()
            for b in range(NBUF):
                c = g * NBUF + b
                out_copy(c, b).wait()

                @pl.when(g < NGRP - 1)
                def _():
                    gather_copy(c + NBUF, b).start()

            return carry

        lax.fori_loop(0, NGRP, group, 0)

    return gather_k


_gather_rows = _make_gather()


def _proj_body(g_ref, w_ref, o_ref):
    r = lax.dot_general(
        g_ref[...], w_ref[...],
        dimension_numbers=(((1,), (1,)), ((), ())),
        preferred_element_type=jnp.float32,
    )
    o_ref[...] = r.reshape(PROJ_BB, HIST, OUT)


def _project(gathered, W):
    return pl.pallas_call(
        _proj_body,
        grid=(BATCH // PROJ_BB,),
        in_specs=[
            pl.BlockSpec((PROJ_BB * HIST, EMB), lambda i: (i, 0)),
            pl.BlockSpec((OUT, EMB), lambda i: (0, 0)),
        ],
        out_specs=pl.BlockSpec((PROJ_BB, HIST, OUT), lambda i: (i, 0, 0)),
        out_shape=jax.ShapeDtypeStruct((BATCH, HIST, OUT), jnp.float32),
    )(gathered, W)


def kernel(seqs, emb_table, W):
    idx = seqs.astype(jnp.int32).reshape(NW, NCHUNK, CH)
    gathered = _gather_rows(emb_table, idx)
    return _project(gathered, W)


# speedup vs baseline: 25.0081x; 1.5978x over previous
import functools

import jax
import jax.numpy as jnp
from jax import lax
from jax.experimental import pallas as pl
from jax.experimental.pallas import tpu as pltpu
from jax.experimental.pallas import tpu_sc as plsc

VOCAB = 1000000
EMB = 128
OUT = 64
B_TOTAL = 16384 * 50

NC, NS = 2, 16
NW = NC * NS
CH = 128
B_PER_W = B_TOTAL // NW
NCHUNK = B_PER_W // CH

BATCH = 16384
HIST = 50
PROJ_BB = 4096


NBUF = 4
NGRP = NCHUNK // NBUF


def _make_gather():
    mesh = plsc.VectorSubcoreMesh(core_axis_name="c", subcore_axis_name="s")

    @functools.partial(
        pl.kernel,
        mesh=mesh,
        out_type=jax.ShapeDtypeStruct((B_TOTAL, EMB), jnp.float32),
        scratch_types=[
            pltpu.VMEM((NCHUNK, CH), jnp.int32),
            pltpu.VMEM((NBUF, CH, EMB), jnp.float32),
        ] + [pltpu.SemaphoreType.DMA] * (2 * NBUF),
    )
    def gather_k(tab_hbm, idx_hbm, out_hbm, idx_v, rows_v, *sems):
        gsems, osems = sems[:NBUF], sems[NBUF:]
        wid = lax.axis_index("s") * NC + lax.axis_index("c")
        base = wid * B_PER_W
        pltpu.sync_copy(idx_hbm.at[wid], idx_v)

        def gather_copy(c, b):
            return pltpu.make_async_copy(
                tab_hbm.at[idx_v.at[c]], rows_v.at[b], gsems[b])

        def out_copy(c, b):
            return pltpu.make_async_copy(
                rows_v.at[b], out_hbm.at[pl.ds(base + c * CH, CH)], osems[b])

        for b in range(NBUF):
            gather_copy(b, b).start()

        def group(g, carry):
            for b in range(NBUF):
                c = g * NBUF + b
                gather_copy(c, b).wait()
                out_copy(c, b).start()
            for b in range(NBUF):
                c = g * NBUF + b
                out_copy(c, b).wait()

                @pl.when(g < NGRP - 1)
                def _():
                    gather_copy(c + NBUF, b).start()

            return carry

        lax.fori_loop(0, NGRP, group, 0)

    return gather_k


_gather_rows = _make_gather()


def _proj_body(w_ref, g_ref, o_ref):
    r = lax.dot_general(
        w_ref[...], g_ref[...],
        dimension_numbers=(((1,), (1,)), ((), ())),
        preferred_element_type=jnp.float32,
    )
    o_ref[...] = r.reshape(1, OUT, PROJ_BB)


def _project(gathered, W):
    nb = BATCH // PROJ_BB
    return pl.pallas_call(
        _proj_body,
        grid=(HIST, nb),
        in_specs=[
            pl.BlockSpec((OUT, EMB), lambda l, i: (0, 0)),
            pl.BlockSpec((PROJ_BB, EMB), lambda l, i: (l * nb + i, 0)),
        ],
        out_specs=pl.BlockSpec((1, OUT, PROJ_BB), lambda l, i: (l, 0, i)),
        out_shape=jax.ShapeDtypeStruct((HIST, OUT, BATCH), jnp.float32),
    )(W, gathered)


def kernel(seqs, emb_table, W):
    idx = seqs.astype(jnp.int32).T.reshape(NW, NCHUNK, CH)
    gathered = _gather_rows(emb_table, idx)
    return _project(gathered, W).transpose(2, 0, 1)


# speedup vs baseline: 26.6157x; 1.0643x over previous
import functools

import jax
import jax.numpy as jnp
from jax import lax
from jax.experimental import pallas as pl
from jax.experimental.pallas import tpu as pltpu
from jax.experimental.pallas import tpu_sc as plsc

VOCAB = 1000000
EMB = 128
OUT = 64
BATCH = 16384
HIST = 50
B_TOTAL = BATCH * HIST

NSLICE = 2
HIST_S = HIST // NSLICE
B_SLICE = B_TOTAL // NSLICE

NC, NS = 2, 16
NW = NC * NS
CH = 128
B_PER_W = B_SLICE // NW
NCHUNK = B_PER_W // CH
NBUF = 4
NGRP = NCHUNK // NBUF

PROJ_BB = 4096


def _make_gather():
    mesh = plsc.VectorSubcoreMesh(core_axis_name="c", subcore_axis_name="s")

    @functools.partial(
        pl.kernel,
        mesh=mesh,
        out_type=jax.ShapeDtypeStruct((B_SLICE, EMB), jnp.float32),
        scratch_types=[
            pltpu.VMEM((NCHUNK, CH), jnp.int32),
            pltpu.VMEM((NBUF, CH, EMB), jnp.float32),
        ] + [pltpu.SemaphoreType.DMA] * (2 * NBUF),
    )
    def gather_k(tab_hbm, idx_hbm, out_hbm, idx_v, rows_v, *sems):
        gsems, osems = sems[:NBUF], sems[NBUF:]
        wid = lax.axis_index("s") * NC + lax.axis_index("c")
        base = wid * B_PER_W
        pltpu.sync_copy(idx_hbm.at[wid], idx_v)

        def gather_copy(c, b):
            return pltpu.make_async_copy(
                tab_hbm.at[idx_v.at[c]], rows_v.at[b], gsems[b])

        def out_copy(c, b):
            return pltpu.make_async_copy(
                rows_v.at[b], out_hbm.at[pl.ds(base + c * CH, CH)], osems[b])

        for b in range(NBUF):
            gather_copy(b, b).start()

        def group(g, carry):
            for b in range(NBUF):
                c = g * NBUF + b
                gather_copy(c, b).wait()
                out_copy(c, b).start()
            for b in range(NBUF):
                c = g * NBUF + b
                out_copy(c, b).wait()

                @pl.when(g < NGRP - 1)
                def _():
                    gather_copy(c + NBUF, b).start()

            return carry

        lax.fori_loop(0, NGRP, group, 0)

    return gather_k


_gather_rows = _make_gather()


def _proj_body(w_ref, g_ref, o_ref):
    r = lax.dot_general(
        w_ref[...], g_ref[...],
        dimension_numbers=(((1,), (1,)), ((), ())),
        preferred_element_type=jnp.float32,
    )
    o_ref[...] = r.reshape(1, OUT, PROJ_BB)


def _project_slice(gathered, W, l_off, prev=None):
    nb = BATCH // PROJ_BB
    in_specs = [
        pl.BlockSpec((OUT, EMB), lambda l, i: (0, 0)),
        pl.BlockSpec((PROJ_BB, EMB), lambda l, i: (l * nb + i, 0)),
    ]
    args = [W, gathered]
    kwargs = {}
    if prev is not None:
        in_specs.append(pl.BlockSpec(memory_space=pl.ANY))
        args.append(prev)
        kwargs["input_output_aliases"] = {2: 0}

    def body(w_ref, g_ref, *refs):
        _proj_body(w_ref, g_ref, refs[-1])

    return pl.pallas_call(
        body,
        grid=(HIST_S, nb),
        in_specs=in_specs,
        out_specs=pl.BlockSpec(
            (1, OUT, PROJ_BB), lambda l, i: (l + l_off, 0, i)),
        out_shape=jax.ShapeDtypeStruct((HIST, OUT, BATCH), jnp.float32),
        **kwargs,
    )(*args)


def kernel(seqs, emb_table, W):
    idx = seqs.astype(jnp.int32).T.reshape(NSLICE, NW, NCHUNK, CH)
    out = None
    for s in range(NSLICE):
        gathered = _gather_rows(emb_table, idx[s])
        out = _project_slice(gathered, W, s * HIST_S, out)
    return out.transpose(2, 0, 1)


# speedup vs baseline: 27.3400x; 1.0272x over previous
import functools

import jax
import jax.numpy as jnp
from jax import lax
from jax.experimental import pallas as pl
from jax.experimental.pallas import tpu as pltpu
from jax.experimental.pallas import tpu_sc as plsc

VOCAB = 1000000
EMB = 128
OUT = 64
BATCH = 16384
HIST = 50
B_TOTAL = BATCH * HIST

NSLICE = 5
HIST_S = HIST // NSLICE
B_SLICE = B_TOTAL // NSLICE

NC, NS = 2, 16
NW = NC * NS
CH = 128
B_PER_W = B_SLICE // NW
NCHUNK = B_PER_W // CH
NBUF = 4
NGRP = NCHUNK // NBUF

PROJ_BB = 4096


def _make_gather():
    mesh = plsc.VectorSubcoreMesh(core_axis_name="c", subcore_axis_name="s")

    @functools.partial(
        pl.kernel,
        mesh=mesh,
        out_type=jax.ShapeDtypeStruct((B_SLICE, EMB), jnp.float32),
        scratch_types=[
            pltpu.VMEM((NCHUNK, CH), jnp.int32),
            pltpu.VMEM((NBUF, CH, EMB), jnp.float32),
        ] + [pltpu.SemaphoreType.DMA] * (2 * NBUF),
    )
    def gather_k(tab_hbm, idx_hbm, out_hbm, idx_v, rows_v, *sems):
        gsems, osems = sems[:NBUF], sems[NBUF:]
        wid = lax.axis_index("s") * NC + lax.axis_index("c")
        base = wid * B_PER_W
        pltpu.sync_copy(idx_hbm.at[wid], idx_v)

        def gather_copy(c, b):
            return pltpu.make_async_copy(
                tab_hbm.at[idx_v.at[c]], rows_v.at[b], gsems[b])

        def out_copy(c, b):
            return pltpu.make_async_copy(
                rows_v.at[b], out_hbm.at[pl.ds(base + c * CH, CH)], osems[b])

        for b in range(NBUF):
            gather_copy(b, b).start()

        def group(g, carry):
            for b in range(NBUF):
                c = g * NBUF + b
                gather_copy(c, b).wait()
                out_copy(c, b).start()
            for b in range(NBUF):
                c = g * NBUF + b
                out_copy(c, b).wait()

                @pl.when(g < NGRP - 1)
                def _():
                    gather_copy(c + NBUF, b).start()

            return carry

        lax.fori_loop(0, NGRP, group, 0)

    return gather_k


_gather_rows = _make_gather()


def _proj_body(w_ref, g_ref, o_ref):
    r = lax.dot_general(
        w_ref[...], g_ref[...],
        dimension_numbers=(((1,), (1,)), ((), ())),
        preferred_element_type=jnp.float32,
    )
    o_ref[...] = r.reshape(1, OUT, PROJ_BB)


def _project_slice(gathered, W, l_off, prev=None):
    nb = BATCH // PROJ_BB
    in_specs = [
        pl.BlockSpec((OUT, EMB), lambda l, i: (0, 0)),
        pl.BlockSpec((PROJ_BB, EMB), lambda l, i: (l * nb + i, 0)),
    ]
    args = [W, gathered]
    kwargs = {}
    if prev is not None:
        in_specs.append(pl.BlockSpec(memory_space=pl.ANY))
        args.append(prev)
        kwargs["input_output_aliases"] = {2: 0}

    def body(w_ref, g_ref, *refs):
        _proj_body(w_ref, g_ref, refs[-1])

    return pl.pallas_call(
        body,
        grid=(HIST_S, nb),
        in_specs=in_specs,
        out_specs=pl.BlockSpec(
            (1, OUT, PROJ_BB), lambda l, i: (l + l_off, 0, i)),
        out_shape=jax.ShapeDtypeStruct((HIST, OUT, BATCH), jnp.float32),
        **kwargs,
    )(*args)


def kernel(seqs, emb_table, W):
    idx = seqs.astype(jnp.int32).T.reshape(NSLICE, NW, NCHUNK, CH)
    out = None
    for s in range(NSLICE):
        gathered = _gather_rows(emb_table, idx[s])
        out = _project_slice(gathered, W, s * HIST_S, out)
    return out.transpose(2, 0, 1)
